# baseline (speedup 1.0000x reference)
import functools

import jax
import jax.numpy as jnp
from jax import lax
from jax.experimental import pallas as pl
from jax.experimental.pallas import tpu as pltpu
from jax.experimental.pallas import tpu_sc as plsc

NC = 2
NS = 16
NW = NC * NS
L = 16

_CONST_CACHE = {}


def _col_indices(rows, cols, ns):
    def compute():
        perm_key = jax.random.key(42)
        u = jax.random.uniform(perm_key, (rows, cols))
        return jnp.argsort(u, axis=1)[:, :ns].astype(jnp.int32)

    k = (rows, cols, ns)
    if k not in _CONST_CACHE:
        try:
            with jax.ensure_compile_time_eval():
                idx = compute()
            _CONST_CACHE[k] = jax.block_until_ready(idx)
        except Exception:
            return compute()
    return _CONST_CACHE[k]


def _rsqrt16(v):
    i = lax.bitcast_convert_type(v, jnp.int32)
    y = lax.bitcast_convert_type(jnp.int32(0x5F3759DF) - (i >> 1), jnp.float32)
    for _ in range(3):
        y = y * (1.5 - 0.5 * v * y * y)
    return y


def _make_sc_call(rows, cols, ns):
    assert rows % NW == 0
    rows_per_w = rows // NW
    R = 8
    assert rows_per_w % (2 * R) == 0
    n_blocks = rows_per_w // R
    n_pairs = n_blocks // 2
    nch = ns // L

    mesh = plsc.VectorSubcoreMesh(core_axis_name="c", subcore_axis_name="s")

    @functools.partial(
        pl.kernel,
        mesh=mesh,
        out_type=jax.ShapeDtypeStruct((rows, cols), jnp.float32),
        scratch_types=[
            pltpu.VMEM((R, cols), jnp.float32),
            pltpu.VMEM((R, cols), jnp.float32),
            pltpu.VMEM((R, ns), jnp.int32),
            pltpu.VMEM((R, ns), jnp.int32),
            pltpu.VMEM((R, cols), jnp.float32),
            pltpu.VMEM((ns,), jnp.float32),
            pltpu.VMEM((ns,), jnp.float32),
            pltpu.SemaphoreType.DMA,
            pltpu.SemaphoreType.DMA,
            pltpu.SemaphoreType.DMA,
            pltpu.SemaphoreType.DMA,
        ],
        compiler_params=pltpu.CompilerParams(needs_layout_passes=False),
    )
    def sc_kernel(x_hbm, idx_hbm, g_hbm, b_hbm, out_hbm,
                  xb0, xb1, ib0, ib1, ob, gv, bv, sx0, sx1, si0, si1):
        c = lax.axis_index("c")
        s = lax.axis_index("s")
        wid = s * NC + c
        row0 = wid * rows_per_w

        pltpu.sync_copy(g_hbm, gv)
        pltpu.sync_copy(b_hbm, bv)

        zero16 = jnp.zeros((L,), jnp.float32)

        def start_in(bk, xb, ib, sx, si):
            r0 = row0 + bk * R
            pltpu.async_copy(x_hbm.at[pl.ds(r0, R)], xb, sx)
            pltpu.async_copy(idx_hbm.at[pl.ds(r0, R)], ib, si)

        def wait_in(bk, xb, ib, sx, si):
            r0 = row0 + bk * R
            pltpu.make_async_copy(x_hbm.at[pl.ds(r0, R)], xb, sx).wait()
            pltpu.make_async_copy(idx_hbm.at[pl.ds(r0, R)], ib, si).wait()

        def _zrow(r, _):
            def _zchunk(t, _):
                ob[r, pl.ds(t * L, L)] = zero16
                return 0
            return lax.fori_loop(0, cols // L, _zchunk, 0)

        lax.fori_loop(0, R, _zrow, 0)

        def process_block(bk, xb, ib):
            r0 = row0 + bk * R

            @plsc.parallel_loop(0, R, 1, unroll=4)
            def row_body(rl):
                row_iv = jnp.full((L,), rl, jnp.int32)
                vals = []
                ssum = zero16
                ssq = zero16
                for j in range(nch):
                    iv = ib[rl, pl.ds(j * L, L)]
                    v = plsc.load_gather(xb, [row_iv, iv])
                    vals.append(v)
                    ssum = ssum + v
                    ssq = ssq + v * v
                inv_n = 1.0 / ns
                mu = jnp.sum(ssum) * inv_n
                var = jnp.sum(ssq) * inv_n - mu * mu
                mu_v = jnp.full((L,), mu)
                rinv = _rsqrt16(jnp.full((L,), var + 1e-5))
                evals = []
                acc = zero16
                for j in range(nch):
                    xn = (vals[j] - mu_v) * rinv * gv[pl.ds(j * L, L)] + bv[pl.ds(j * L, L)]
                    e = jnp.exp(xn)
                    evals.append(e)
                    acc = acc + e
                rsd = _rsqrt16(jnp.full((L,), jnp.sum(acc)))
                rd_v = rsd * rsd
                for j in range(nch):
                    iv = ib[rl, pl.ds(j * L, L)]
                    plsc.store_scatter(ob, [row_iv, iv], evals[j] * rd_v)

            pltpu.sync_copy(ob, out_hbm.at[pl.ds(r0, R)])

            @plsc.parallel_loop(0, R, 1, unroll=2)
            def rz_body(rl):
                row_iv = jnp.full((L,), rl, jnp.int32)
                for j in range(nch):
                    iv = ib[rl, pl.ds(j * L, L)]
                    plsc.store_scatter(ob, [row_iv, iv], zero16)

        start_in(0, xb0, ib0, sx0, si0)

        def pair_body(t, _):
            b0 = 2 * t
            b1 = b0 + 1
            b2 = b0 + 2
            wait_in(b0, xb0, ib0, sx0, si0)
            start_in(b1, xb1, ib1, sx1, si1)
            process_block(b0, xb0, ib0)
            wait_in(b1, xb1, ib1, sx1, si1)

            @pl.when(b2 < n_blocks)
            def _():
                start_in(b2, xb0, ib0, sx0, si0)

            process_block(b1, xb1, ib1)
            return 0

        lax.fori_loop(0, n_pairs, pair_body, 0)

    return sc_kernel


def kernel(x, gamma, beta):
    shape = x.shape
    xf = x.reshape(-1, shape[-1])
    rows, cols = xf.shape
    ns = gamma.shape[0]
    idx = _col_indices(rows, cols, ns)
    out = _make_sc_call(rows, cols, ns)(xf, idx, gamma, beta)
    return out.reshape(shape)


# speedup vs baseline: 1.5886x; 1.5886x over previous
import functools

import jax
import jax.numpy as jnp
from jax import lax
from jax.experimental import pallas as pl
from jax.experimental.pallas import tpu as pltpu
from jax.experimental.pallas import tpu_sc as plsc

NC = 2
NS = 16
NW = NC * NS
L = 16

_CONST_CACHE = {}


def _col_indices(rows, cols, ns):
    def compute():
        perm_key = jax.random.key(42)
        u = jax.random.uniform(perm_key, (rows, cols))
        return jnp.argsort(u, axis=1)[:, :ns].astype(jnp.int32)

    k = (rows, cols, ns)
    if k not in _CONST_CACHE:
        try:
            with jax.ensure_compile_time_eval():
                idx = compute()
            _CONST_CACHE[k] = jax.block_until_ready(idx)
        except Exception:
            return compute()
    return _CONST_CACHE[k]


def _rsqrt16(v):
    i = lax.bitcast_convert_type(v, jnp.int32)
    y = lax.bitcast_convert_type(jnp.int32(0x5F3759DF) - (i >> 1), jnp.float32)
    for _ in range(3):
        y = y * (1.5 - 0.5 * v * y * y)
    return y


def _make_sc_call(rows, cols, ns):
    assert rows % NW == 0
    rows_per_w = rows // NW
    R = 8
    assert rows_per_w % (2 * R) == 0
    n_blocks = rows_per_w // R
    n_pairs = n_blocks // 2
    nch = ns // L

    mesh = plsc.VectorSubcoreMesh(core_axis_name="c", subcore_axis_name="s")

    @functools.partial(
        pl.kernel,
        mesh=mesh,
        out_type=jax.ShapeDtypeStruct((rows, cols), jnp.float32),
        scratch_types=[
            pltpu.VMEM((R, cols), jnp.float32),
            pltpu.VMEM((R, cols), jnp.float32),
            pltpu.VMEM((R, ns), jnp.int32),
            pltpu.VMEM((R, ns), jnp.int32),
            pltpu.VMEM((R, cols), jnp.float32),
            pltpu.VMEM((ns,), jnp.float32),
            pltpu.VMEM((ns,), jnp.float32),
            pltpu.SemaphoreType.DMA,
            pltpu.SemaphoreType.DMA,
            pltpu.SemaphoreType.DMA,
            pltpu.SemaphoreType.DMA,
        ],
        compiler_params=pltpu.CompilerParams(needs_layout_passes=False),
    )
    def sc_kernel(x_hbm, idx_hbm, g_hbm, b_hbm, out_hbm,
                  xb0, xb1, ib0, ib1, ob, gv, bv, sx0, sx1, si0, si1):
        c = lax.axis_index("c")
        s = lax.axis_index("s")
        wid = s * NC + c
        row0 = wid * rows_per_w

        pltpu.sync_copy(g_hbm, gv)
        pltpu.sync_copy(b_hbm, bv)

        zero16 = jnp.zeros((L,), jnp.float32)

        def start_in(bk, xb, ib, sx, si):
            r0 = row0 + bk * R
            pltpu.async_copy(x_hbm.at[pl.ds(r0, R)], xb, sx)
            pltpu.async_copy(idx_hbm.at[pl.ds(r0, R)], ib, si)

        def wait_in(bk, xb, ib, sx, si):
            r0 = row0 + bk * R
            pltpu.make_async_copy(x_hbm.at[pl.ds(r0, R)], xb, sx).wait()
            pltpu.make_async_copy(idx_hbm.at[pl.ds(r0, R)], ib, si).wait()

        def _zrow(r, _):
            def _zchunk(t, _):
                ob[r, pl.ds(t * L, L)] = zero16
                return 0
            return lax.fori_loop(0, cols // L, _zchunk, 0)

        lax.fori_loop(0, R, _zrow, 0)

        def process_block(bk, xb, ib):
            r0 = row0 + bk * R

            @plsc.parallel_loop(0, R, 1, unroll=2)
            def row_body(rl):
                row_iv = jnp.full((L,), rl, jnp.int32)
                vals = []
                ssum = zero16
                ssq = zero16
                for j in range(nch):
                    iv = ib[rl, pl.ds(j * L, L)]
                    v = plsc.load_gather(xb, [row_iv, iv])
                    vals.append(v)
                    ssum = ssum + v
                    ssq = ssq + v * v
                inv_n = 1.0 / ns
                mu = jnp.sum(ssum) * inv_n
                var = jnp.sum(ssq) * inv_n - mu * mu
                mu_v = jnp.full((L,), mu)
                rinv = _rsqrt16(jnp.full((L,), var + 1e-5))
                evals = []
                acc = zero16
                for j in range(nch):
                    xn = (vals[j] - mu_v) * rinv * gv[pl.ds(j * L, L)] + bv[pl.ds(j * L, L)]
                    e = jnp.exp(xn)
                    evals.append(e)
                    acc = acc + e
                rsd = _rsqrt16(jnp.full((L,), jnp.sum(acc)))
                rd_v = rsd * rsd
                for j in range(nch):
                    iv = ib[rl, pl.ds(j * L, L)]
                    plsc.store_scatter(ob, [row_iv, iv], evals[j] * rd_v)

            pltpu.sync_copy(ob, out_hbm.at[pl.ds(r0, R)])

            @plsc.parallel_loop(0, R, 1, unroll=2)
            def rz_body(rl):
                row_iv = jnp.full((L,), rl, jnp.int32)
                for j in range(nch):
                    iv = ib[rl, pl.ds(j * L, L)]
                    plsc.store_scatter(ob, [row_iv, iv], zero16)

        start_in(0, xb0, ib0, sx0, si0)

        def pair_body(t, _):
            b0 = 2 * t
            b1 = b0 + 1
            b2 = b0 + 2
            wait_in(b0, xb0, ib0, sx0, si0)
            start_in(b1, xb1, ib1, sx1, si1)
            process_block(b0, xb0, ib0)
            wait_in(b1, xb1, ib1, sx1, si1)

            @pl.when(b2 < n_blocks)
            def _():
                start_in(b2, xb0, ib0, sx0, si0)

            process_block(b1, xb1, ib1)
            return 0

        lax.fori_loop(0, n_pairs, pair_body, 0)

    return sc_kernel


def kernel(x, gamma, beta):
    shape = x.shape
    xf = x.reshape(-1, shape[-1])
    rows, cols = xf.shape
    ns = gamma.shape[0]
    idx = _col_indices(rows, cols, ns)
    out = _make_sc_call(rows, cols, ns)(xf, idx, gamma, beta)
    return out.reshape(shape)
